# Initial kernel scaffold; baseline (speedup 1.0000x reference)
import jax
import jax.numpy as jnp
from jax.experimental import pallas as pl


def kernel(x, edge_index, batch, y, w_rel1, b_rel1, w_root1, p1, w_rel2, b_rel2, w_root2, p2, w_rel3, b_rel3, w_root3, p3, lin1_w, lin1_b, lin2_w, lin2_b, lin3_w, lin3_b):
    raise NotImplementedError("write your pallas kernel here")


# speedup vs baseline: 3.5865x; 3.5865x over previous
import functools

import jax
import jax.numpy as jnp
import numpy as np
from jax.experimental import pallas as pl
from jax.experimental.pallas import tpu as pltpu

N_NODES = 100000
NUM_GRAPHS = 64
RATIO = 0.7
NEG = -1e30

_BLK = 512
N_PAD = ((N_NODES + _BLK - 1) // _BLK) * _BLK


def _conv_body(agg_ref, h_ref, wrel_ref, b_ref, wroot_ref, mask_ref, out_ref):
    out = (agg_ref[...] @ wrel_ref[...] + b_ref[...]
           + h_ref[...] @ wroot_ref[...])
    out_ref[...] = jnp.maximum(out, 0.0) * mask_ref[...]


def _conv_relu(agg, h, w_rel, b_rel, w_root, mask):
    n, fin = h.shape
    fout = w_rel.shape[0]
    grid = (n // _BLK,)
    return pl.pallas_call(
        _conv_body,
        grid=grid,
        in_specs=[
            pl.BlockSpec((_BLK, fin), lambda i: (i, 0)),
            pl.BlockSpec((_BLK, fin), lambda i: (i, 0)),
            pl.BlockSpec((fin, fout), lambda i: (0, 0)),
            pl.BlockSpec((1, fout), lambda i: (0, 0)),
            pl.BlockSpec((fin, fout), lambda i: (0, 0)),
            pl.BlockSpec((_BLK, 1), lambda i: (i, 0)),
        ],
        out_specs=pl.BlockSpec((_BLK, fout), lambda i: (i, 0)),
        out_shape=jax.ShapeDtypeStruct((n, fout), jnp.float32),
    )(agg, h, w_rel.T, b_rel[None, :], w_root.T, mask[:, None])


def _topk_pool(x, batch, mask, p, starts):
    score = jnp.tanh((x @ p) / jnp.linalg.norm(p))
    sort_score = jnp.where(mask > 0, score, -5.0)
    key = batch.astype(jnp.float32) * 10.0 - sort_score
    order = jnp.argsort(key)
    batch_sorted = batch[order]
    rank = jnp.arange(batch.shape[0], dtype=jnp.int32) - starts[batch_sorted]
    valid_counts = jax.ops.segment_sum(mask, batch, num_segments=NUM_GRAPHS)
    k = jnp.ceil(RATIO * valid_counts).astype(jnp.int32)
    keep_sorted = (rank < k[batch_sorted]).astype(x.dtype)
    new_mask = jnp.zeros_like(mask).at[order].set(keep_sorted)
    x_new = x * score[:, None] * new_mask[:, None]
    return x_new, new_mask


def _readout(x, batch, mask):
    counts = jax.ops.segment_sum(mask, batch, num_segments=NUM_GRAPHS)
    denom = jnp.maximum(counts, 1.0)
    mean = jax.ops.segment_sum(x * mask[:, None], batch, num_segments=NUM_GRAPHS) / denom[:, None]
    mx = jax.ops.segment_max(jnp.where(mask[:, None] > 0, x, NEG), batch, num_segments=NUM_GRAPHS)
    mx = jnp.where(counts[:, None] > 0, mx, 0.0)
    mn = jax.ops.segment_min(jnp.where(mask[:, None] > 0, x, -NEG), batch, num_segments=NUM_GRAPHS)
    mn = jnp.where(counts[:, None] > 0, mn, 0.0)
    return jnp.concatenate([mx, mean, mn], axis=1)


def kernel(x, edge_index, batch, y, w_rel1, b_rel1, w_root1, p1, w_rel2, b_rel2, w_root2, p2, w_rel3, b_rel3, w_root3, p3, lin1_w, lin1_b, lin2_w, lin2_b, lin3_w, lin3_b):
    feat = x[:, :16]
    src, dst = edge_index[0], edge_index[1]
    n = feat.shape[0]
    mask = jnp.ones((n,), jnp.float32)
    counts_all = jnp.bincount(batch, length=NUM_GRAPHS)
    starts = jnp.concatenate([jnp.zeros((1,), counts_all.dtype), jnp.cumsum(counts_all)[:-1]]).astype(jnp.int32)

    def pad_nodes(a):
        return jnp.pad(a, [(0, N_PAD - N_NODES)] + [(0, 0)] * (a.ndim - 1))

    batch_p = jnp.pad(batch, (0, N_PAD - N_NODES), constant_values=NUM_GRAPHS - 1)

    agg = jnp.zeros_like(feat).at[dst].add(feat[src])
    h = _conv_relu(pad_nodes(agg), pad_nodes(feat), w_rel1, b_rel1, w_root1,
                   pad_nodes(mask))[:n]
    h, mask = _topk_pool(h, batch, mask, p1, starts)
    x1 = _readout(h, batch, mask)

    agg = jnp.zeros_like(h).at[dst].add(h[src])
    h = _conv_relu(pad_nodes(agg), pad_nodes(h), w_rel2, b_rel2, w_root2,
                   pad_nodes(mask))[:n]
    h, mask = _topk_pool(h, batch, mask, p2, starts)
    x2 = _readout(h, batch, mask)

    agg = jnp.zeros_like(h).at[dst].add(h[src])
    h = _conv_relu(pad_nodes(agg), pad_nodes(h), w_rel3, b_rel3, w_root3,
                   pad_nodes(mask))[:n]
    h, mask = _topk_pool(h, batch, mask, p3, starts)
    x3 = _readout(h, batch, mask)

    z = jnp.concatenate([x1, x2, x3], axis=1)
    z = jax.nn.relu(z @ lin1_w.T + lin1_b)
    z = jax.nn.relu(z @ lin2_w.T + lin2_b)
    return z @ lin3_w.T + lin3_b


# speedup vs baseline: 8.3194x; 2.3196x over previous
import functools

import jax
import jax.numpy as jnp
import numpy as np
from jax import lax
from jax.experimental import pallas as pl
from jax.experimental.pallas import tpu as pltpu
from jax.experimental.pallas import tpu_sc as plsc

N_NODES = 100000
N_EDGES = 1600000
NUM_GRAPHS = 64
RATIO = 0.7
NEG = -1e30

_BLK = 512
N_PAD = ((N_NODES + _BLK - 1) // _BLK) * _BLK
_NTILES = 32
_EB = 128
E_PAD = _NTILES * _EB * 392
_TILE_E = E_PAD // _NTILES
_NG = _TILE_E // _EB // 2
_NR = N_PAD // 16
_ZR = 784


def _make_sc_agg(C):
    mesh = plsc.VectorSubcoreMesh(core_axis_name="c", subcore_axis_name="s")

    @functools.partial(
        pl.kernel,
        mesh=mesh,
        compiler_params=pltpu.CompilerParams(use_tc_tiling_on_sc=False),
        out_type=jax.ShapeDtypeStruct((2, C, N_PAD, 16), jnp.float32),
        scratch_types=[
            pltpu.VMEM_SHARED((N_PAD, 16), jnp.float32),
            pltpu.VMEM((_ZR, 16), jnp.float32),
            pltpu.VMEM((2, _EB), jnp.int32),
            pltpu.VMEM((2, _EB), jnp.int32),
            pltpu.VMEM((2, _EB), jnp.int32),
            pltpu.VMEM((2, _EB, 16), jnp.float32),
            pltpu.SemaphoreType.DMA,
            pltpu.SemaphoreType.DMA,
            pltpu.SemaphoreType.DMA,
            pltpu.SemaphoreType.DMA,
            pltpu.SemaphoreType.DMA,
            pltpu.SemaphoreType.DMA,
        ],
    )
    def k(h2, srcp, dstp, zrows, out, shared, zbuf, sidx, didx, gidx, rows,
          sems0, sems1, semd0, semd1, semg0, semg1):
        sems = (sems0, sems1)
        semd = (semd0, semd1)
        semg = (semg0, semg1)
        cc = lax.axis_index("c")
        ss = lax.axis_index("s")
        e0 = (cc * 16 + ss) * _TILE_E
        row0 = ss * _NR
        pltpu.sync_copy(zrows, zbuf)

        for chunk in range(C):
            for z in range(_NR // _ZR):
                pltpu.sync_copy(zbuf, shared.at[pl.ds(row0 + z * _ZR, _ZR)])
            plsc.subcore_barrier()

            for j in range(2):
                pltpu.make_async_copy(
                    srcp.at[pl.ds(e0 + j * _EB, _EB)], sidx.at[j], sems[j]
                ).start()

            def group(g, carry):
                for j in range(2):
                    b0 = e0 + (2 * g + j) * _EB
                    pltpu.make_async_copy(
                        srcp.at[pl.ds(e0, _EB)], sidx.at[j], sems[j]
                    ).wait()
                    for q in range(_EB // 16):
                        sl = pl.ds(q * 16, 16)
                        gidx[j, sl] = sidx[j, sl] * C + chunk
                    pltpu.make_async_copy(h2.at[gidx.at[j]], rows.at[j], semg[j]).start()
                    pltpu.make_async_copy(dstp.at[pl.ds(b0, _EB)], didx.at[j], semd[j]).start()

                @pl.when(g < _NG - 1)
                def _prefetch():
                    for j in range(2):
                        b0 = e0 + (2 * (g + 1) + j) * _EB
                        pltpu.make_async_copy(
                            srcp.at[pl.ds(b0, _EB)], sidx.at[j], sems[j]
                        ).start()

                for j in range(2):
                    pltpu.make_async_copy(h2.at[gidx.at[j]], rows.at[j], semg[j]).wait()
                    pltpu.make_async_copy(
                        dstp.at[pl.ds(e0, _EB)], didx.at[j], semd[j]
                    ).wait()
                    pltpu.sync_copy(rows.at[j], shared.at[didx.at[j]], add=True)
                return carry

            lax.fori_loop(0, _NG, group, 0)
            plsc.subcore_barrier()

            pltpu.sync_copy(
                shared.at[pl.ds(row0, _NR)],
                out.at[cc, chunk, pl.ds(row0, _NR)],
            )

    return k


_sc_agg_1 = _make_sc_agg(1)
_sc_agg_8 = _make_sc_agg(8)


def _conv_body(p0_ref, p1_ref, h_ref, wrel_ref, b_ref, wroot_ref, mask_ref, out_ref):
    agg = p0_ref[...] + p1_ref[...]
    out = agg @ wrel_ref[...] + b_ref[...] + h_ref[...] @ wroot_ref[...]
    out_ref[...] = jnp.maximum(out, 0.0) * mask_ref[...]


def _conv_relu(p0, p1, h, w_rel, b_rel, w_root, mask):
    n, fin = h.shape
    fout = w_rel.shape[0]
    grid = (n // _BLK,)
    return pl.pallas_call(
        _conv_body,
        grid=grid,
        in_specs=[
            pl.BlockSpec((_BLK, fin), lambda i: (i, 0)),
            pl.BlockSpec((_BLK, fin), lambda i: (i, 0)),
            pl.BlockSpec((_BLK, fin), lambda i: (i, 0)),
            pl.BlockSpec((fin, fout), lambda i: (0, 0)),
            pl.BlockSpec((1, fout), lambda i: (0, 0)),
            pl.BlockSpec((fin, fout), lambda i: (0, 0)),
            pl.BlockSpec((_BLK, 1), lambda i: (i, 0)),
        ],
        out_specs=pl.BlockSpec((_BLK, fout), lambda i: (i, 0)),
        out_shape=jax.ShapeDtypeStruct((n, fout), jnp.float32),
    )(p0, p1, h, w_rel.T, b_rel[None, :], w_root.T, mask[:, None])


def _sc_conv(h_pad, srcp, dstp, zrows, w_rel, b_rel, w_root, mask_pad, C):
    h2 = h_pad.reshape(N_PAD * C, 16)
    agg_fn = _sc_agg_1 if C == 1 else _sc_agg_8
    aggp = agg_fn(h2, srcp, dstp, zrows)
    aggp = aggp.transpose(0, 2, 1, 3).reshape(2, N_PAD, C * 16)
    return _conv_relu(aggp[0], aggp[1], h_pad, w_rel, b_rel, w_root, mask_pad)


def _topk_pool(x, batch, mask, p, starts):
    score = jnp.tanh((x @ p) / jnp.linalg.norm(p))
    sort_score = jnp.where(mask > 0, score, -5.0)
    key = batch.astype(jnp.float32) * 10.0 - sort_score
    order = jnp.argsort(key)
    batch_sorted = batch[order]
    rank = jnp.arange(batch.shape[0], dtype=jnp.int32) - starts[batch_sorted]
    valid_counts = jax.ops.segment_sum(mask, batch, num_segments=NUM_GRAPHS)
    k = jnp.ceil(RATIO * valid_counts).astype(jnp.int32)
    keep_sorted = (rank < k[batch_sorted]).astype(x.dtype)
    new_mask = jnp.zeros_like(mask).at[order].set(keep_sorted)
    x_new = x * score[:, None] * new_mask[:, None]
    return x_new, new_mask


def _readout(x, batch, mask):
    counts = jax.ops.segment_sum(mask, batch, num_segments=NUM_GRAPHS)
    denom = jnp.maximum(counts, 1.0)
    mean = jax.ops.segment_sum(x * mask[:, None], batch, num_segments=NUM_GRAPHS) / denom[:, None]
    mx = jax.ops.segment_max(jnp.where(mask[:, None] > 0, x, NEG), batch, num_segments=NUM_GRAPHS)
    mx = jnp.where(counts[:, None] > 0, mx, 0.0)
    mn = jax.ops.segment_min(jnp.where(mask[:, None] > 0, x, -NEG), batch, num_segments=NUM_GRAPHS)
    mn = jnp.where(counts[:, None] > 0, mn, 0.0)
    return jnp.concatenate([mx, mean, mn], axis=1)


def kernel(x, edge_index, batch, y, w_rel1, b_rel1, w_root1, p1, w_rel2, b_rel2, w_root2, p2, w_rel3, b_rel3, w_root3, p3, lin1_w, lin1_b, lin2_w, lin2_b, lin3_w, lin3_b):
    feat = x[:, :16]
    src, dst = edge_index[0], edge_index[1]
    n = feat.shape[0]
    mask = jnp.ones((n,), jnp.float32)
    counts_all = jnp.bincount(batch, length=NUM_GRAPHS)
    starts = jnp.concatenate([jnp.zeros((1,), counts_all.dtype), jnp.cumsum(counts_all)[:-1]]).astype(jnp.int32)

    def pad_nodes(a):
        return jnp.pad(a, [(0, N_PAD - N_NODES)] + [(0, 0)] * (a.ndim - 1))

    srcp = jnp.pad(src, (0, E_PAD - N_EDGES), constant_values=N_NODES)
    dstp = jnp.pad(dst, (0, E_PAD - N_EDGES), constant_values=N_NODES)
    zrows = jnp.zeros((_ZR, 16), jnp.float32)

    h = _sc_conv(pad_nodes(feat), srcp, dstp, zrows, w_rel1, b_rel1, w_root1,
                 pad_nodes(mask), 1)[:n]
    h, mask = _topk_pool(h, batch, mask, p1, starts)
    x1 = _readout(h, batch, mask)

    h = _sc_conv(pad_nodes(h), srcp, dstp, zrows, w_rel2, b_rel2, w_root2,
                 pad_nodes(mask), 8)[:n]
    h, mask = _topk_pool(h, batch, mask, p2, starts)
    x2 = _readout(h, batch, mask)

    h = _sc_conv(pad_nodes(h), srcp, dstp, zrows, w_rel3, b_rel3, w_root3,
                 pad_nodes(mask), 8)[:n]
    h, mask = _topk_pool(h, batch, mask, p3, starts)
    x3 = _readout(h, batch, mask)

    z = jnp.concatenate([x1, x2, x3], axis=1)
    z = jax.nn.relu(z @ lin1_w.T + lin1_b)
    z = jax.nn.relu(z @ lin2_w.T + lin2_b)
    return z @ lin3_w.T + lin3_b


# speedup vs baseline: 9.1194x; 1.0962x over previous
import functools

import jax
import jax.numpy as jnp
import numpy as np
from jax import lax
from jax.experimental import pallas as pl
from jax.experimental.pallas import tpu as pltpu
from jax.experimental.pallas import tpu_sc as plsc

N_NODES = 100000
N_EDGES = 1600000
NUM_GRAPHS = 64
RATIO = 0.7
NEG = -1e30

_BLK = 512
N_PAD = ((N_NODES + _BLK - 1) // _BLK) * _BLK
_NTILES = 32
_EB = 128
E_PAD = _NTILES * _EB * 392
_TILE_E = E_PAD // _NTILES
_NSLOT = 4
_NG = _TILE_E // _EB // _NSLOT
_NR = N_PAD // 16
_ZR = 784


def _make_sc_agg(C):
    mesh = plsc.VectorSubcoreMesh(core_axis_name="c", subcore_axis_name="s")

    @functools.partial(
        pl.kernel,
        mesh=mesh,
        compiler_params=pltpu.CompilerParams(use_tc_tiling_on_sc=False),
        out_type=jax.ShapeDtypeStruct((2, C, N_PAD, 16), jnp.float32),
        scratch_types=[
            pltpu.VMEM_SHARED((N_PAD, 16), jnp.float32),
            pltpu.VMEM((_ZR, 16), jnp.float32),
            pltpu.VMEM((_NSLOT, _EB), jnp.int32),
            pltpu.VMEM((_NSLOT, _EB), jnp.int32),
            pltpu.VMEM((_NSLOT, _EB), jnp.int32),
            pltpu.VMEM((_NSLOT, _EB, 16), jnp.float32),
        ] + [pltpu.SemaphoreType.DMA] * (3 * _NSLOT),
    )
    def k(h2, srcp, dstp, zrows, out, shared, zbuf, sidx, didx, gidx, rows,
          *allsems):
        sems = allsems[0:_NSLOT]
        semd = allsems[_NSLOT:2 * _NSLOT]
        semg = allsems[2 * _NSLOT:3 * _NSLOT]
        cc = lax.axis_index("c")
        ss = lax.axis_index("s")
        e0 = (cc * 16 + ss) * _TILE_E
        row0 = ss * _NR
        pltpu.sync_copy(zrows, zbuf)

        for chunk in range(C):
            for z in range(_NR // _ZR):
                pltpu.sync_copy(zbuf, shared.at[pl.ds(row0 + z * _ZR, _ZR)])
            plsc.subcore_barrier()

            for j in range(_NSLOT):
                pltpu.make_async_copy(
                    srcp.at[pl.ds(e0 + j * _EB, _EB)], sidx.at[j], sems[j]
                ).start()

            def group(g, carry):
                for j in range(_NSLOT):
                    b0 = e0 + (_NSLOT * g + j) * _EB
                    pltpu.make_async_copy(
                        srcp.at[pl.ds(e0, _EB)], sidx.at[j], sems[j]
                    ).wait()
                    for q in range(_EB // 16):
                        sl = pl.ds(q * 16, 16)
                        gidx[j, sl] = sidx[j, sl] * C + chunk
                    pltpu.make_async_copy(h2.at[gidx.at[j]], rows.at[j], semg[j]).start()
                    pltpu.make_async_copy(dstp.at[pl.ds(b0, _EB)], didx.at[j], semd[j]).start()

                @pl.when(g < _NG - 1)
                def _prefetch():
                    for j in range(_NSLOT):
                        b0 = e0 + (_NSLOT * (g + 1) + j) * _EB
                        pltpu.make_async_copy(
                            srcp.at[pl.ds(b0, _EB)], sidx.at[j], sems[j]
                        ).start()

                for j in range(_NSLOT):
                    pltpu.make_async_copy(h2.at[gidx.at[j]], rows.at[j], semg[j]).wait()
                    pltpu.make_async_copy(
                        dstp.at[pl.ds(e0, _EB)], didx.at[j], semd[j]
                    ).wait()
                    pltpu.sync_copy(rows.at[j], shared.at[didx.at[j]], add=True)
                return carry

            lax.fori_loop(0, _NG, group, 0)
            plsc.subcore_barrier()

            pltpu.sync_copy(
                shared.at[pl.ds(row0, _NR)],
                out.at[cc, chunk, pl.ds(row0, _NR)],
            )

    return k


_sc_agg_1 = _make_sc_agg(1)
_sc_agg_8 = _make_sc_agg(8)


def _conv_body(p0_ref, p1_ref, h_ref, wrel_ref, b_ref, wroot_ref, mask_ref, out_ref):
    agg = p0_ref[...] + p1_ref[...]
    out = agg @ wrel_ref[...] + b_ref[...] + h_ref[...] @ wroot_ref[...]
    out_ref[...] = jnp.maximum(out, 0.0) * mask_ref[...]


def _conv_relu(p0, p1, h, w_rel, b_rel, w_root, mask):
    n, fin = h.shape
    fout = w_rel.shape[0]
    grid = (n // _BLK,)
    return pl.pallas_call(
        _conv_body,
        grid=grid,
        in_specs=[
            pl.BlockSpec((_BLK, fin), lambda i: (i, 0)),
            pl.BlockSpec((_BLK, fin), lambda i: (i, 0)),
            pl.BlockSpec((_BLK, fin), lambda i: (i, 0)),
            pl.BlockSpec((fin, fout), lambda i: (0, 0)),
            pl.BlockSpec((1, fout), lambda i: (0, 0)),
            pl.BlockSpec((fin, fout), lambda i: (0, 0)),
            pl.BlockSpec((_BLK, 1), lambda i: (i, 0)),
        ],
        out_specs=pl.BlockSpec((_BLK, fout), lambda i: (i, 0)),
        out_shape=jax.ShapeDtypeStruct((n, fout), jnp.float32),
    )(p0, p1, h, w_rel.T, b_rel[None, :], w_root.T, mask[:, None])


def _sc_conv(h_pad, srcp, dstp, zrows, w_rel, b_rel, w_root, mask_pad, C):
    h2 = h_pad.reshape(N_PAD * C, 16)
    agg_fn = _sc_agg_1 if C == 1 else _sc_agg_8
    aggp = agg_fn(h2, srcp, dstp, zrows)
    aggp = aggp.transpose(0, 2, 1, 3).reshape(2, N_PAD, C * 16)
    return _conv_relu(aggp[0], aggp[1], h_pad, w_rel, b_rel, w_root, mask_pad)


def _topk_pool(x, batch, mask, p, starts):
    score = jnp.tanh((x @ p) / jnp.linalg.norm(p))
    sort_score = jnp.where(mask > 0, score, -5.0)
    key = batch.astype(jnp.float32) * 10.0 - sort_score
    order = jnp.argsort(key)
    batch_sorted = batch[order]
    rank = jnp.arange(batch.shape[0], dtype=jnp.int32) - starts[batch_sorted]
    valid_counts = jax.ops.segment_sum(mask, batch, num_segments=NUM_GRAPHS)
    k = jnp.ceil(RATIO * valid_counts).astype(jnp.int32)
    keep_sorted = (rank < k[batch_sorted]).astype(x.dtype)
    new_mask = jnp.zeros_like(mask).at[order].set(keep_sorted)
    x_new = x * score[:, None] * new_mask[:, None]
    return x_new, new_mask


def _readout(x, batch, mask):
    counts = jax.ops.segment_sum(mask, batch, num_segments=NUM_GRAPHS)
    denom = jnp.maximum(counts, 1.0)
    mean = jax.ops.segment_sum(x * mask[:, None], batch, num_segments=NUM_GRAPHS) / denom[:, None]
    mx = jax.ops.segment_max(jnp.where(mask[:, None] > 0, x, NEG), batch, num_segments=NUM_GRAPHS)
    mx = jnp.where(counts[:, None] > 0, mx, 0.0)
    mn = jax.ops.segment_min(jnp.where(mask[:, None] > 0, x, -NEG), batch, num_segments=NUM_GRAPHS)
    mn = jnp.where(counts[:, None] > 0, mn, 0.0)
    return jnp.concatenate([mx, mean, mn], axis=1)


def kernel(x, edge_index, batch, y, w_rel1, b_rel1, w_root1, p1, w_rel2, b_rel2, w_root2, p2, w_rel3, b_rel3, w_root3, p3, lin1_w, lin1_b, lin2_w, lin2_b, lin3_w, lin3_b):
    feat = x[:, :16]
    src, dst = edge_index[0], edge_index[1]
    n = feat.shape[0]
    mask = jnp.ones((n,), jnp.float32)
    counts_all = jnp.bincount(batch, length=NUM_GRAPHS)
    starts = jnp.concatenate([jnp.zeros((1,), counts_all.dtype), jnp.cumsum(counts_all)[:-1]]).astype(jnp.int32)

    def pad_nodes(a):
        return jnp.pad(a, [(0, N_PAD - N_NODES)] + [(0, 0)] * (a.ndim - 1))

    srcp = jnp.pad(src, (0, E_PAD - N_EDGES), constant_values=N_NODES)
    dstp = jnp.pad(dst, (0, E_PAD - N_EDGES), constant_values=N_NODES)
    zrows = jnp.zeros((_ZR, 16), jnp.float32)

    h = _sc_conv(pad_nodes(feat), srcp, dstp, zrows, w_rel1, b_rel1, w_root1,
                 pad_nodes(mask), 1)[:n]
    h, mask = _topk_pool(h, batch, mask, p1, starts)
    x1 = _readout(h, batch, mask)

    h = _sc_conv(pad_nodes(h), srcp, dstp, zrows, w_rel2, b_rel2, w_root2,
                 pad_nodes(mask), 8)[:n]
    h, mask = _topk_pool(h, batch, mask, p2, starts)
    x2 = _readout(h, batch, mask)

    h = _sc_conv(pad_nodes(h), srcp, dstp, zrows, w_rel3, b_rel3, w_root3,
                 pad_nodes(mask), 8)[:n]
    h, mask = _topk_pool(h, batch, mask, p3, starts)
    x3 = _readout(h, batch, mask)

    z = jnp.concatenate([x1, x2, x3], axis=1)
    z = jax.nn.relu(z @ lin1_w.T + lin1_b)
    z = jax.nn.relu(z @ lin2_w.T + lin2_b)
    return z @ lin3_w.T + lin3_b
